# baseline (speedup 1.0000x reference)
import functools

import jax
import jax.numpy as jnp
from jax import lax
from jax.experimental import pallas as pl
from jax.experimental.pallas import tpu as pltpu
from jax.experimental.pallas import tpu_sc as plsc

VOCAB = 23236
DIM = 1728
DIM_PAD = 1792
BATCH = 1024
SEQ = 50
SEQ_PAD = 56
NP = BATCH * SEQ_PAD
NC, NS = 2, 16
NW = NC * NS
ROWS_PER_W = NP // NW
CH = 8
NBUF = 8
NCHUNK = ROWS_PER_W // CH

PAD_BR = 256
DEPAD_NB = 8


def _emb_body(table_hbm, idx_hbm, out_hbm, idx_v, rows_v, gsem, ssem):
    wid = lax.axis_index("s") * NC + lax.axis_index("c")
    base = wid * ROWS_PER_W

    pltpu.sync_copy(idx_hbm.at[wid], idx_v)

    def idx_at(c):
        return idx_v.at[c // 16, pl.ds((c % 16) * CH, CH)]

    def gather(c, slot):
        return pltpu.async_copy(
            table_hbm.at[idx_at(c)], rows_v.at[slot], gsem.at[slot])

    def scatter(c, slot):
        return pltpu.async_copy(
            rows_v.at[slot], out_hbm.at[pl.ds(base + c * CH, CH)], ssem.at[slot])

    def wait_gather(slot):
        pltpu.make_async_copy(
            table_hbm.at[idx_at(0)], rows_v.at[slot], gsem.at[slot]).wait()

    def wait_scatter(c, slot):
        pltpu.make_async_copy(
            rows_v.at[slot], out_hbm.at[pl.ds(base + c * CH, CH)],
            ssem.at[slot]).wait()

    K = NBUF // 2
    for p in range(K):
        gather(p, p)

    def step(c, slot, first, last):
        wait_gather(slot)
        scatter(c, slot)
        if not first:
            wait_scatter(c - K, (c + K) % NBUF)
        if not last:
            gather(c + K, (c + K) % NBUF)

    for c in range(NBUF):
        step(c, c, c < K, False)

    def octet(t, _):
        c0 = NBUF * t
        for i in range(NBUF):
            step(c0 + i, i, False, False)
        return _

    lax.fori_loop(1, NCHUNK // NBUF - 1, octet, 0)

    for i in range(NBUF):
        c = NCHUNK - NBUF + i
        step(c, i, False, i >= NBUF - K)
    for i in range(K):
        c = NCHUNK - K + i
        wait_scatter(c, c % NBUF)


def _pad_body(w_ref, o_ref):
    o_ref[...] = jnp.concatenate(
        [w_ref[...], jnp.zeros((PAD_BR, DIM_PAD - DIM), jnp.float32)], axis=1)


def _depad_body(i_ref, o_ref):
    for i in range(DEPAD_NB):
        o_ref[i] = i_ref[pl.ds(i * SEQ_PAD, SEQ), :DIM]


@jax.jit
def _emb(weight, idx):
    wpad = pl.pallas_call(
        _pad_body,
        grid=(pl.cdiv(VOCAB, PAD_BR),),
        in_specs=[pl.BlockSpec((PAD_BR, DIM), lambda g: (g, 0))],
        out_specs=pl.BlockSpec((PAD_BR, DIM_PAD), lambda g: (g, 0)),
        out_shape=jax.ShapeDtypeStruct((VOCAB, DIM_PAD), jnp.float32),
    )(weight)

    mesh = plsc.VectorSubcoreMesh(
        core_axis_name="c", subcore_axis_name="s", num_cores=NC, num_subcores=NS)
    f = pl.kernel(
        _emb_body,
        out_type=jax.ShapeDtypeStruct((NP, DIM_PAD), jnp.float32),
        mesh=mesh,
        scratch_types=[
            pltpu.VMEM((NCHUNK // 16, 128), jnp.int32),
            pltpu.VMEM((NBUF, CH, DIM_PAD), jnp.float32),
            pltpu.SemaphoreType.DMA((NBUF,)),
            pltpu.SemaphoreType.DMA((NBUF,)),
        ],
    )
    gathered = f(wpad, idx)

    return pl.pallas_call(
        _depad_body,
        grid=(BATCH // DEPAD_NB,),
        in_specs=[pl.BlockSpec((DEPAD_NB * SEQ_PAD, DIM_PAD), lambda g: (g, 0))],
        out_specs=pl.BlockSpec((DEPAD_NB, SEQ, DIM), lambda g: (g, 0, 0)),
        out_shape=jax.ShapeDtypeStruct((BATCH, SEQ, DIM), jnp.float32),
    )(gathered)


def kernel(input_ids, weight):
    idx = jnp.pad(input_ids, ((0, 0), (0, SEQ_PAD - SEQ)))
    return _emb(weight, idx.reshape(NW, NCHUNK // 16, 128))
